# baseline (speedup 1.0000x reference)
import functools

import jax
import jax.numpy as jnp
from jax import lax
from jax.experimental import pallas as pl
from jax.experimental.pallas import tpu as pltpu
from jax.experimental.pallas import tpu_sc as plsc

N = 16384
C = 1000
NS = 16
ROWS_PER_SUB = N // NS
CHUNKS = ROWS_PER_SUB // 128
LANE = 16


def _body(prob_hbm, tgt_hbm, rew_hbm, out_hbm, part_hbm,
          tgt_v, rew_v, val_v, acc_v, red_v, out_v, sem):
    sid = lax.axis_index("s")
    base = sid * ROWS_PER_SUB

    pltpu.sync_copy(tgt_hbm.at[pl.ds(base, ROWS_PER_SUB)], tgt_v)
    pltpu.sync_copy(rew_hbm.at[pl.ds(base, ROWS_PER_SUB)], rew_v)

    lane = lax.iota(jnp.int32, LANE)
    copies = []
    for j in range(CHUNKS):
        for k in range(8):
            c = j * 8 + k
            t = tgt_v[pl.ds(c * LANE, LANE)]
            idx = (base + c * LANE + lane) * C + t
            cp = pltpu.make_async_copy(
                prob_hbm.at[idx], val_v.at[j, pl.ds(k * LANE, LANE)], sem)
            cp.start()
            copies.append(cp)
    for cp in copies:
        cp.wait()

    acc = jnp.zeros((LANE,), jnp.float32)
    for j in range(CHUNKS):
        for k in range(8):
            c = j * 8 + k
            v = val_v[j, pl.ds(k * LANE, LANE)]
            r = rew_v[pl.ds(c * LANE, LANE)]
            acc = acc + jnp.exp(v) * r
    acc_v[...] = acc

    pltpu.sync_copy(acc_v, part_hbm.at[sid])
    plsc.subcore_barrier()

    @pl.when(sid == 0)
    def _():
        pltpu.sync_copy(part_hbm, red_v)
        tot = jnp.zeros((LANE,), jnp.float32)
        for s in range(NS):
            tot = tot + red_v[s, :]
        tot = tot * (1.0 / N)
        mean = tot[0]
        for i in range(1, LANE):
            mean = mean + tot[i]
        out_v[...] = jnp.broadcast_to(mean, (LANE,))
        pltpu.sync_copy(out_v, out_hbm)


@jax.jit
def _ganloss_sc(prob_flat, target, reward):
    mesh = plsc.VectorSubcoreMesh(core_axis_name="c", subcore_axis_name="s",
                                  num_cores=1)
    k = functools.partial(
        pl.kernel,
        mesh=mesh,
        out_type=(jax.ShapeDtypeStruct((LANE,), jnp.float32),
                  jax.ShapeDtypeStruct((NS, LANE), jnp.float32)),
        scratch_types=[
            pltpu.VMEM((ROWS_PER_SUB,), jnp.int32),
            pltpu.VMEM((ROWS_PER_SUB,), jnp.float32),
            pltpu.VMEM((CHUNKS, 128), jnp.float32),
            pltpu.VMEM((LANE,), jnp.float32),
            pltpu.VMEM((NS, LANE), jnp.float32),
            pltpu.VMEM((LANE,), jnp.float32),
            pltpu.SemaphoreType.DMA,
        ],
    )(_body)
    return k(prob_flat, target, reward)


def kernel(prob, target, reward):
    out, _ = _ganloss_sc(prob.reshape(-1), target.astype(jnp.int32),
                         reward.astype(jnp.float32))
    return out[0]


# speedup vs baseline: 1.0018x; 1.0018x over previous
import functools

import jax
import jax.numpy as jnp
from jax import lax
from jax.experimental import pallas as pl
from jax.experimental.pallas import tpu as pltpu
from jax.experimental.pallas import tpu_sc as plsc

N = 16384
C = 1000
NS = 16
ROWS_PER_SUB = N // NS
CHUNKS = ROWS_PER_SUB // 128
LANE = 16


def _body(prob_hbm, tgt_hbm, rew_hbm, out_hbm, part_hbm,
          tgt_v, rew_v, idx_v, val_v, acc_v, red_v, out_v, sem):
    sid = lax.axis_index("s")
    base = sid * ROWS_PER_SUB

    pltpu.sync_copy(tgt_hbm.at[pl.ds(base, ROWS_PER_SUB)], tgt_v)
    pltpu.sync_copy(rew_hbm.at[pl.ds(base, ROWS_PER_SUB)], rew_v)

    lane = lax.iota(jnp.int32, LANE)
    for j in range(CHUNKS):
        for k in range(8):
            c = j * 8 + k
            t = tgt_v[pl.ds(c * LANE, LANE)]
            idx_v[j, pl.ds(k * LANE, LANE)] = (base + c * LANE + lane) * C + t
    copies = [
        pltpu.make_async_copy(prob_hbm.at[idx_v.at[j]], val_v.at[j], sem)
        for j in range(CHUNKS)
    ]
    for cp in copies:
        cp.start()
    for cp in copies:
        cp.wait()

    acc = jnp.zeros((LANE,), jnp.float32)
    for j in range(CHUNKS):
        for k in range(8):
            c = j * 8 + k
            v = val_v[j, pl.ds(k * LANE, LANE)]
            r = rew_v[pl.ds(c * LANE, LANE)]
            acc = acc + jnp.exp(v) * r
    acc_v[...] = acc

    pltpu.sync_copy(acc_v, part_hbm.at[sid])
    plsc.subcore_barrier()

    @pl.when(sid == 0)
    def _():
        pltpu.sync_copy(part_hbm, red_v)
        tot = jnp.zeros((LANE,), jnp.float32)
        for s in range(NS):
            tot = tot + red_v[s, :]
        tot = tot * (1.0 / N)
        mean = tot[0]
        for i in range(1, LANE):
            mean = mean + tot[i]
        out_v[...] = jnp.broadcast_to(mean, (LANE,))
        pltpu.sync_copy(out_v, out_hbm)


@jax.jit
def _ganloss_sc(prob_flat, target, reward):
    mesh = plsc.VectorSubcoreMesh(core_axis_name="c", subcore_axis_name="s",
                                  num_cores=1)
    k = functools.partial(
        pl.kernel,
        mesh=mesh,
        out_type=(jax.ShapeDtypeStruct((LANE,), jnp.float32),
                  jax.ShapeDtypeStruct((NS, LANE), jnp.float32)),
        scratch_types=[
            pltpu.VMEM((ROWS_PER_SUB,), jnp.int32),
            pltpu.VMEM((ROWS_PER_SUB,), jnp.float32),
            pltpu.VMEM((CHUNKS, 128), jnp.int32),
            pltpu.VMEM((CHUNKS, 128), jnp.float32),
            pltpu.VMEM((LANE,), jnp.float32),
            pltpu.VMEM((NS, LANE), jnp.float32),
            pltpu.VMEM((LANE,), jnp.float32),
            pltpu.SemaphoreType.DMA,
        ],
    )(_body)
    return k(prob_flat, target, reward)


def kernel(prob, target, reward):
    out, _ = _ganloss_sc(prob.reshape(-1), target.astype(jnp.int32),
                         reward.astype(jnp.float32))
    return out[0]


# speedup vs baseline: 4.0277x; 4.0206x over previous
import functools

import jax
import jax.numpy as jnp
from jax import lax
from jax.experimental import pallas as pl
from jax.experimental.pallas import tpu as pltpu
from jax.experimental.pallas import tpu_sc as plsc

N = 16384
C = 1000
NS = 16
ROWS_PER_SUB = N // NS
BLK = 128
BLOCKS = ROWS_PER_SUB // BLK
LANE = 16


def _body(probT_hbm, tgt_hbm, rew_hbm, out_hbm, part_hbm,
          tgt_v, rew_v, val_v, acc_v, red_v, out_v, sem_a, sem_b):
    sid = lax.axis_index("s")
    base = sid * ROWS_PER_SUB

    pltpu.sync_copy(tgt_hbm.at[pl.ds(base, ROWS_PER_SUB)], tgt_v)
    pltpu.sync_copy(rew_hbm.at[pl.ds(base, ROWS_PER_SUB)], rew_v)

    sems = (sem_a, sem_b)

    def fire(b):
        idx_ref = tgt_v.at[pl.ds(b * BLK, BLK)]
        cp = pltpu.make_async_copy(
            probT_hbm.at[idx_ref, pl.ds(base + b * BLK, BLK)],
            val_v.at[b % 2], sems[b % 2])
        cp.start()
        return cp

    iota = lax.iota(jnp.int32, LANE)
    onehot = [jnp.where(iota == i, 1.0, 0.0) for i in range(LANE)]
    cps = [fire(0), fire(1)]
    acc = jnp.zeros((LANE,), jnp.float32)
    for b in range(BLOCKS):
        cps[b].wait()
        for k in range(BLK // LANE):
            diag = jnp.zeros((LANE,), jnp.float32)
            for i in range(LANE):
                row = val_v[b % 2, k * LANE + i, pl.ds(k * LANE, LANE)]
                diag = diag + row * onehot[i]
            r = rew_v[pl.ds(b * BLK + k * LANE, LANE)]
            acc = acc + jnp.exp(diag) * r
        if b + 2 < BLOCKS:
            cps.append(fire(b + 2))
    acc_v[...] = acc

    pltpu.sync_copy(acc_v, part_hbm.at[sid])
    plsc.subcore_barrier()

    @pl.when(sid == 0)
    def _():
        pltpu.sync_copy(part_hbm, red_v)
        tot = jnp.zeros((LANE,), jnp.float32)
        for s in range(NS):
            tot = tot + red_v[s, :]
        tot = tot * (1.0 / N)
        mean = tot[0]
        for i in range(1, LANE):
            mean = mean + tot[i]
        out_v[...] = jnp.broadcast_to(mean, (LANE,))
        pltpu.sync_copy(out_v, out_hbm)


@jax.jit
def _ganloss_sc(probT, target, reward):
    mesh = plsc.VectorSubcoreMesh(core_axis_name="c", subcore_axis_name="s",
                                  num_cores=1)
    k = functools.partial(
        pl.kernel,
        mesh=mesh,
        out_type=(jax.ShapeDtypeStruct((LANE,), jnp.float32),
                  jax.ShapeDtypeStruct((NS, LANE), jnp.float32)),
        scratch_types=[
            pltpu.VMEM((ROWS_PER_SUB,), jnp.int32),
            pltpu.VMEM((ROWS_PER_SUB,), jnp.float32),
            pltpu.VMEM((2, BLK, BLK), jnp.float32),
            pltpu.VMEM((LANE,), jnp.float32),
            pltpu.VMEM((NS, LANE), jnp.float32),
            pltpu.VMEM((LANE,), jnp.float32),
            pltpu.SemaphoreType.DMA,
            pltpu.SemaphoreType.DMA,
        ],
    )(_body)
    return k(probT, target, reward)


def kernel(prob, target, reward):
    out, _ = _ganloss_sc(prob.T, target.astype(jnp.int32),
                         reward.astype(jnp.float32))
    return out[0]


# speedup vs baseline: 4.2082x; 1.0448x over previous
import functools

import jax
import jax.numpy as jnp
from jax import lax
from jax.experimental import pallas as pl
from jax.experimental.pallas import tpu as pltpu
from jax.experimental.pallas import tpu_sc as plsc

N = 16384
C = 1000
NS = 16
ROWS_PER_SUB = N // NS
BLK = 128
BLOCKS = ROWS_PER_SUB // BLK
LANE = 16


def _body(probT_hbm, tgt_hbm, rew_hbm, out_hbm, part_hbm,
          tgt_v, rew_v, val_v, acc_v, red_v, out_v,
          sem_s, sem_a, sem_b, sem_c, sem_d):
    sid = lax.axis_index("s")
    base = sid * ROWS_PER_SUB

    cp_t = pltpu.make_async_copy(
        tgt_hbm.at[pl.ds(base, ROWS_PER_SUB)], tgt_v, sem_s)
    cp_r = pltpu.make_async_copy(
        rew_hbm.at[pl.ds(base, ROWS_PER_SUB)], rew_v, sem_s)
    cp_t.start()
    cp_r.start()
    cp_t.wait()

    sems = (sem_a, sem_b, sem_c, sem_d)
    NBUF = len(sems)

    def fire(b):
        idx_ref = tgt_v.at[pl.ds(b * BLK, BLK)]
        cp = pltpu.make_async_copy(
            probT_hbm.at[idx_ref, pl.ds(base + b * BLK, BLK)],
            val_v.at[b % NBUF], sems[b % NBUF])
        cp.start()
        return cp

    iota = lax.iota(jnp.int32, LANE)
    onehot = [jnp.where(iota == i, 1.0, 0.0) for i in range(LANE)]
    cps = [fire(b) for b in range(min(NBUF, BLOCKS))]
    cp_r.wait()
    acc = jnp.zeros((LANE,), jnp.float32)
    for b in range(BLOCKS):
        cps[b].wait()
        for k in range(BLK // LANE):
            d0 = jnp.zeros((LANE,), jnp.float32)
            d1 = jnp.zeros((LANE,), jnp.float32)
            for i in range(0, LANE, 2):
                d0 = d0 + val_v[b % NBUF, k * LANE + i,
                                pl.ds(k * LANE, LANE)] * onehot[i]
                d1 = d1 + val_v[b % NBUF, k * LANE + i + 1,
                                pl.ds(k * LANE, LANE)] * onehot[i + 1]
            r = rew_v[pl.ds(b * BLK + k * LANE, LANE)]
            acc = acc + jnp.exp(d0 + d1) * r
        if b + NBUF < BLOCKS:
            cps.append(fire(b + NBUF))
    acc_v[...] = acc

    pltpu.sync_copy(acc_v, part_hbm.at[sid])
    plsc.subcore_barrier()

    @pl.when(sid == 0)
    def _():
        pltpu.sync_copy(part_hbm, red_v)
        tot = jnp.zeros((LANE,), jnp.float32)
        for s in range(NS):
            tot = tot + red_v[s, :]
        tot = tot * (1.0 / N)
        mean = tot[0]
        for i in range(1, LANE):
            mean = mean + tot[i]
        out_v[...] = jnp.broadcast_to(mean, (LANE,))
        pltpu.sync_copy(out_v, out_hbm)


@jax.jit
def _ganloss_sc(probT, target, reward):
    mesh = plsc.VectorSubcoreMesh(core_axis_name="c", subcore_axis_name="s",
                                  num_cores=1)
    k = functools.partial(
        pl.kernel,
        mesh=mesh,
        out_type=(jax.ShapeDtypeStruct((LANE,), jnp.float32),
                  jax.ShapeDtypeStruct((NS, LANE), jnp.float32)),
        scratch_types=[
            pltpu.VMEM((ROWS_PER_SUB,), jnp.int32),
            pltpu.VMEM((ROWS_PER_SUB,), jnp.float32),
            pltpu.VMEM((4, BLK, BLK), jnp.float32),
            pltpu.VMEM((LANE,), jnp.float32),
            pltpu.VMEM((NS, LANE), jnp.float32),
            pltpu.VMEM((LANE,), jnp.float32),
            pltpu.SemaphoreType.DMA,
            pltpu.SemaphoreType.DMA,
            pltpu.SemaphoreType.DMA,
            pltpu.SemaphoreType.DMA,
            pltpu.SemaphoreType.DMA,
        ],
    )(_body)
    return k(probT, target, reward)


def kernel(prob, target, reward):
    out, _ = _ganloss_sc(prob.T, target.astype(jnp.int32),
                         reward.astype(jnp.float32))
    return out[0]


# speedup vs baseline: 4.8890x; 1.1618x over previous
import functools

import jax
import jax.numpy as jnp
from jax import lax
from jax.experimental import pallas as pl
from jax.experimental.pallas import tpu as pltpu
from jax.experimental.pallas import tpu_sc as plsc

N = 16384
C = 1000
NC = 2
NS = 16
NW = NC * NS
ROWS_PER_SUB = N // NW
BLK = 128
BLOCKS = ROWS_PER_SUB // BLK
LANE = 16


def _body(probT_hbm, tgt_hbm, rew_hbm, part_hbm,
          tgt_v, rew_v, val_v, acc_v,
          sem_s, sem_a, sem_b, sem_c, sem_d):
    wid = lax.axis_index("c") * NS + lax.axis_index("s")
    base = wid * ROWS_PER_SUB

    cp_t = pltpu.make_async_copy(
        tgt_hbm.at[pl.ds(base, ROWS_PER_SUB)], tgt_v, sem_s)
    cp_r = pltpu.make_async_copy(
        rew_hbm.at[pl.ds(base, ROWS_PER_SUB)], rew_v, sem_s)
    cp_t.start()
    cp_r.start()
    cp_t.wait()

    sems = (sem_a, sem_b, sem_c, sem_d)
    NBUF = len(sems)

    def fire(b):
        idx_ref = tgt_v.at[pl.ds(b * BLK, BLK)]
        cp = pltpu.make_async_copy(
            probT_hbm.at[idx_ref, pl.ds(base + b * BLK, BLK)],
            val_v.at[b % NBUF], sems[b % NBUF])
        cp.start()
        return cp

    iota = lax.iota(jnp.int32, LANE)
    onehot = [jnp.where(iota == i, 1.0, 0.0) for i in range(LANE)]
    cps = [fire(b) for b in range(min(NBUF, BLOCKS))]
    cp_r.wait()
    acc = jnp.zeros((LANE,), jnp.float32)
    for b in range(BLOCKS):
        cps[b].wait()
        for k in range(BLK // LANE):
            d0 = jnp.zeros((LANE,), jnp.float32)
            d1 = jnp.zeros((LANE,), jnp.float32)
            for i in range(0, LANE, 2):
                d0 = d0 + val_v[b % NBUF, k * LANE + i,
                                pl.ds(k * LANE, LANE)] * onehot[i]
                d1 = d1 + val_v[b % NBUF, k * LANE + i + 1,
                                pl.ds(k * LANE, LANE)] * onehot[i + 1]
            r = rew_v[pl.ds(b * BLK + k * LANE, LANE)]
            acc = acc + jnp.exp(d0 + d1) * r
        if b + NBUF < BLOCKS:
            cps.append(fire(b + NBUF))
    acc_v[...] = acc
    pltpu.sync_copy(acc_v, part_hbm.at[wid])


def _tc_reduce(part_ref, out_ref):
    out_ref[0, 0] = jnp.sum(part_ref[...]) * (1.0 / N)


@jax.jit
def _ganloss_sc(probT, target, reward):
    mesh = plsc.VectorSubcoreMesh(core_axis_name="c", subcore_axis_name="s",
                                  num_cores=NC)
    k = functools.partial(
        pl.kernel,
        mesh=mesh,
        out_type=jax.ShapeDtypeStruct((NW, LANE), jnp.float32),
        scratch_types=[
            pltpu.VMEM((ROWS_PER_SUB,), jnp.int32),
            pltpu.VMEM((ROWS_PER_SUB,), jnp.float32),
            pltpu.VMEM((4, BLK, BLK), jnp.float32),
            pltpu.VMEM((LANE,), jnp.float32),
            pltpu.SemaphoreType.DMA,
            pltpu.SemaphoreType.DMA,
            pltpu.SemaphoreType.DMA,
            pltpu.SemaphoreType.DMA,
            pltpu.SemaphoreType.DMA,
        ],
    )(_body)
    part = k(probT, target, reward)
    out = pl.pallas_call(
        _tc_reduce,
        out_shape=jax.ShapeDtypeStruct((1, 1), jnp.float32),
        out_specs=pl.BlockSpec(memory_space=pltpu.SMEM),
    )(part)
    return out[0, 0]


def kernel(prob, target, reward):
    return _ganloss_sc(prob.T, target.astype(jnp.int32),
                       reward.astype(jnp.float32))
